# baseline (speedup 1.0000x reference)
import functools

import jax
import jax.numpy as jnp
from jax.experimental import pallas as pl
from jax.experimental.pallas import tpu as pltpu


def _tokens(q3):
    bt, d, hw = q3.shape
    q = jnp.transpose(q3, (0, 2, 1)).reshape(bt * hw, d)
    qs2 = jnp.sum(q * q, axis=1, keepdims=True)
    return q / jnp.maximum(jnp.sqrt(qs2), 1e-12)


def _pass1(qt_ref, kt_ref, sm_ref, uq_ref, uo_ref, colmax_ref, colsum_ref,
           comp_ref, sep_ref, *, n_total):
    i = pl.program_id(0)
    nt = pl.num_programs(0)
    kt = kt_ref[...]
    d, m = kt.shape
    bt, _, hw = qt_ref.shape

    qn = _tokens(qt_ref[...])
    t = qn.shape[0]
    qnn = jnp.sum(qn * qn, axis=1, keepdims=True)
    qs = jnp.sum(qn, axis=1, keepdims=True)

    kn2 = jnp.sum(kt * kt, axis=0, keepdims=True)
    ksum = jnp.sum(kt, axis=0, keepdims=True)
    combo = kn2 - 2e-6 * ksum

    score = jnp.dot(qn, kt, preferred_element_type=jnp.float32)

    rmax = jnp.max(score, axis=1, keepdims=True)
    er = jnp.exp(score - rmax)
    rsum = jnp.sum(er, axis=1, keepdims=True)
    smv = er * (1.0 / rsum)
    sm_ref[...] = smv
    cm = jax.lax.dot_general(smv, kt, (((1,), (1,)), ((), ())),
                             preferred_element_type=jnp.float32)

    qn_t = jnp.transpose(qn.reshape(bt, hw, d), (0, 2, 1))
    cm_t = jnp.transpose(cm.reshape(bt, hw, d), (0, 2, 1))
    uq_ref[:, :d, :] = qn_t
    uq_ref[:, d:, :] = cm_t
    uo_ref[...] = cm_t

    oh1 = score == rmax
    kn2g = jnp.sum(jnp.where(oh1, kn2, 0.0), axis=1, keepdims=True)
    ksumg = jnp.sum(jnp.where(oh1, ksum, 0.0), axis=1, keepdims=True)
    masked = jnp.where(oh1, -jnp.inf, score)
    m2 = jnp.max(masked, axis=1, keepdims=True)
    oh2 = masked == m2
    cg2 = jnp.sum(jnp.where(oh2, combo, 0.0), axis=1, keepdims=True)

    base = qnn + 2e-6 * qs + 6.4e-11
    comp_part = jnp.sum(qnn - 2.0 * rmax + kn2g)
    dp = jnp.sqrt(jnp.maximum(base - 2.0 * rmax + kn2g - 2e-6 * ksumg, 0.0))
    dn = jnp.sqrt(jnp.maximum(base - 2.0 * m2 + cg2, 0.0))
    sep_part = jnp.sum(jnp.maximum(dp - dn + 1.0, 0.0))

    eS = er * jnp.exp(rmax)
    ctile_max = jnp.max(score, axis=0, keepdims=True)
    ctile_sum = jnp.sum(eS, axis=0, keepdims=True)

    @pl.when(i == 0)
    def _():
        colmax_ref[...] = jnp.full((1, m), -jnp.inf, jnp.float32)
        colsum_ref[...] = jnp.zeros((1, m), jnp.float32)
        comp_ref[0, 0] = 0.0
        sep_ref[0, 0] = 0.0

    colmax_ref[...] = jnp.maximum(colmax_ref[...], ctile_max)
    colsum_ref[...] = colsum_ref[...] + ctile_sum
    comp_ref[0, 0] = comp_ref[0, 0] + comp_part
    sep_ref[0, 0] = sep_ref[0, 0] + sep_part

    @pl.when(i == nt - 1)
    def _():
        comp_ref[0, 0] = comp_ref[0, 0] / float(n_total * d)
        sep_ref[0, 0] = sep_ref[0, 0] / float(n_total)


def _pass2(qt_ref, kt_ref, keys_ref, colmax_ref, colsum_ref, sq_ref, um_ref):
    i = pl.program_id(0)
    nt = pl.num_programs(0)
    kt = kt_ref[...]
    d, m = kt.shape

    qn = _tokens(qt_ref[...])

    score = jnp.dot(qn, kt, preferred_element_type=jnp.float32)
    e = jnp.exp(score)
    sq_ref[...] = e * (1.0 / colsum_ref[...])

    rmax = jnp.max(score, axis=1, keepdims=True)
    oh1 = score == rmax
    cmaxg = jnp.sum(jnp.where(oh1, colmax_ref[...], 0.0),
                    axis=1, keepdims=True)
    wgt = jnp.exp(rmax - cmaxg)
    ohf = jnp.where(oh1, 1.0, 0.0)
    wq = wgt * qn
    part = jax.lax.dot_general(ohf, wq, (((0,), (0,)), ((), ())),
                               preferred_element_type=jnp.float32)

    @pl.when(i == 0)
    def _():
        um_ref[...] = jnp.zeros((m, d), jnp.float32)

    um_ref[...] = um_ref[...] + part

    @pl.when(i == nt - 1)
    def _():
        um = um_ref[...] + keys_ref[...]
        nrm = jnp.sqrt(jnp.sum(um * um, axis=1, keepdims=True))
        um_ref[...] = um / jnp.maximum(nrm, 1e-12)


def kernel(query, keys):
    b, d, h, w = query.shape
    m = keys.shape[0]
    hw = h * w
    n = b * hw
    q3 = query.reshape(b, d, hw)
    kt = keys.T

    bt = 2
    tile = bt * hw
    nt = n // tile

    sm, uq3, uo3, colmax, colsum, comp, sep = pl.pallas_call(
        functools.partial(_pass1, n_total=n),
        grid=(nt,),
        in_specs=[
            pl.BlockSpec((bt, d, hw), lambda i: (i, 0, 0)),
            pl.BlockSpec((d, m), lambda i: (0, 0)),
        ],
        out_specs=[
            pl.BlockSpec((tile, m), lambda i: (i, 0)),
            pl.BlockSpec((bt, 2 * d, hw), lambda i: (i, 0, 0)),
            pl.BlockSpec((bt, d, hw), lambda i: (i, 0, 0)),
            pl.BlockSpec((1, m), lambda i: (0, 0)),
            pl.BlockSpec((1, m), lambda i: (0, 0)),
            pl.BlockSpec(memory_space=pltpu.SMEM),
            pl.BlockSpec(memory_space=pltpu.SMEM),
        ],
        out_shape=[
            jax.ShapeDtypeStruct((n, m), jnp.float32),
            jax.ShapeDtypeStruct((b, 2 * d, hw), jnp.float32),
            jax.ShapeDtypeStruct((b, d, hw), jnp.float32),
            jax.ShapeDtypeStruct((1, m), jnp.float32),
            jax.ShapeDtypeStruct((1, m), jnp.float32),
            jax.ShapeDtypeStruct((1, 1), jnp.float32),
            jax.ShapeDtypeStruct((1, 1), jnp.float32),
        ],
    )(q3, kt)

    sq, um = pl.pallas_call(
        _pass2,
        grid=(nt,),
        in_specs=[
            pl.BlockSpec((bt, d, hw), lambda i: (i, 0, 0)),
            pl.BlockSpec((d, m), lambda i: (0, 0)),
            pl.BlockSpec((m, d), lambda i: (0, 0)),
            pl.BlockSpec((1, m), lambda i: (0, 0)),
            pl.BlockSpec((1, m), lambda i: (0, 0)),
        ],
        out_specs=[
            pl.BlockSpec((tile, m), lambda i: (i, 0)),
            pl.BlockSpec((m, d), lambda i: (0, 0)),
        ],
        out_shape=[
            jax.ShapeDtypeStruct((n, m), jnp.float32),
            jax.ShapeDtypeStruct((m, d), jnp.float32),
        ],
    )(q3, kt, keys, colmax, colsum)

    uq = uq3.reshape(b, 2 * d, h, w)
    uo = uo3.reshape(b, d, h, w)
    return (uq, uo, um, sq, sm, sep.reshape(()), comp.reshape(()))


# speedup vs baseline: 1.0897x; 1.0897x over previous
import functools

import jax
import jax.numpy as jnp
from jax.experimental import pallas as pl
from jax.experimental.pallas import tpu as pltpu


def _pass1(qt_ref, kt_ref, cat_ref, sm_ref, colmax_ref, colsum_ref,
           comp_ref, sep_ref, *, n_total):
    i = pl.program_id(0)
    nt = pl.num_programs(0)
    q = qt_ref[...]
    kt = kt_ref[...]
    t, d = q.shape
    m = kt.shape[1]

    qs2 = jnp.sum(q * q, axis=1, keepdims=True)
    qn = q / jnp.maximum(jnp.sqrt(qs2), 1e-12)
    qnn = jnp.sum(qn * qn, axis=1, keepdims=True)
    qs = jnp.sum(qn, axis=1, keepdims=True)

    kn2 = jnp.sum(kt * kt, axis=0, keepdims=True)
    ksum = jnp.sum(kt, axis=0, keepdims=True)
    combo = kn2 - 2e-6 * ksum

    score = jnp.dot(qn, kt, preferred_element_type=jnp.float32)

    rmax = jnp.max(score, axis=1, keepdims=True)
    er = jnp.exp(score - rmax)
    rsum = jnp.sum(er, axis=1, keepdims=True)
    smv = er * (1.0 / rsum)
    sm_ref[...] = smv
    cm = jax.lax.dot_general(smv, kt, (((1,), (1,)), ((), ())),
                             preferred_element_type=jnp.float32)
    cat_ref[:, :d] = qn
    cat_ref[:, d:] = cm

    oh1 = score == rmax
    kn2g = jnp.sum(jnp.where(oh1, kn2, 0.0), axis=1, keepdims=True)
    ksumg = jnp.sum(jnp.where(oh1, ksum, 0.0), axis=1, keepdims=True)
    masked = jnp.where(oh1, -jnp.inf, score)
    m2 = jnp.max(masked, axis=1, keepdims=True)
    oh2 = masked == m2
    cg2 = jnp.sum(jnp.where(oh2, combo, 0.0), axis=1, keepdims=True)

    base = qnn + 2e-6 * qs + 6.4e-11
    comp_part = jnp.sum(qnn - 2.0 * rmax + kn2g)
    dp = jnp.sqrt(jnp.maximum(base - 2.0 * rmax + kn2g - 2e-6 * ksumg, 0.0))
    dn = jnp.sqrt(jnp.maximum(base - 2.0 * m2 + cg2, 0.0))
    sep_part = jnp.sum(jnp.maximum(dp - dn + 1.0, 0.0))

    eS = er * jnp.exp(rmax)
    ctile_max = jnp.max(score, axis=0, keepdims=True)
    ctile_sum = jnp.sum(eS, axis=0, keepdims=True)

    @pl.when(i == 0)
    def _():
        colmax_ref[...] = jnp.full((1, m), -jnp.inf, jnp.float32)
        colsum_ref[...] = jnp.zeros((1, m), jnp.float32)
        comp_ref[0, 0] = 0.0
        sep_ref[0, 0] = 0.0

    colmax_ref[...] = jnp.maximum(colmax_ref[...], ctile_max)
    colsum_ref[...] = colsum_ref[...] + ctile_sum
    comp_ref[0, 0] = comp_ref[0, 0] + comp_part
    sep_ref[0, 0] = sep_ref[0, 0] + sep_part

    @pl.when(i == nt - 1)
    def _():
        comp_ref[0, 0] = comp_ref[0, 0] / float(n_total * d)
        sep_ref[0, 0] = sep_ref[0, 0] / float(n_total)


def _pass2(cat_ref, kt_ref, keys_ref, colmax_ref, colsum_ref, sq_ref, um_ref):
    i = pl.program_id(0)
    nt = pl.num_programs(0)
    kt = kt_ref[...]
    d, m = kt.shape
    qn = cat_ref[:, :d]

    score = jnp.dot(qn, kt, preferred_element_type=jnp.float32)
    e = jnp.exp(score)
    sq_ref[...] = e * (1.0 / colsum_ref[...])

    rmax = jnp.max(score, axis=1, keepdims=True)
    oh1 = score == rmax
    cmaxg = jnp.sum(jnp.where(oh1, colmax_ref[...], 0.0),
                    axis=1, keepdims=True)
    wgt = jnp.exp(rmax - cmaxg)
    ohb = jnp.where(oh1, 1.0, 0.0).astype(jnp.bfloat16)
    wq = wgt * qn
    hi = wq.astype(jnp.bfloat16)
    lo = (wq - hi.astype(jnp.float32)).astype(jnp.bfloat16)
    dn_t = (((0,), (0,)), ((), ()))
    part = (jax.lax.dot_general(ohb, hi, dn_t,
                                preferred_element_type=jnp.float32)
            + jax.lax.dot_general(ohb, lo, dn_t,
                                  preferred_element_type=jnp.float32))

    @pl.when(i == 0)
    def _():
        um_ref[...] = jnp.zeros((m, d), jnp.float32)

    um_ref[...] = um_ref[...] + part

    @pl.when(i == nt - 1)
    def _():
        um = um_ref[...] + keys_ref[...]
        nrm = jnp.sqrt(jnp.sum(um * um, axis=1, keepdims=True))
        um_ref[...] = um / jnp.maximum(nrm, 1e-12)


def kernel(query, keys):
    b, d, h, w = query.shape
    m = keys.shape[0]
    n = b * h * w
    qt = jnp.transpose(query, (0, 2, 3, 1)).reshape(n, d)
    kt = keys.T

    tile = 2048
    nt = n // tile

    cat, sm, colmax, colsum, comp, sep = pl.pallas_call(
        functools.partial(_pass1, n_total=n),
        grid=(nt,),
        in_specs=[
            pl.BlockSpec((tile, d), lambda i: (i, 0)),
            pl.BlockSpec((d, m), lambda i: (0, 0)),
        ],
        out_specs=[
            pl.BlockSpec((tile, 2 * d), lambda i: (i, 0)),
            pl.BlockSpec((tile, m), lambda i: (i, 0)),
            pl.BlockSpec((1, m), lambda i: (0, 0)),
            pl.BlockSpec((1, m), lambda i: (0, 0)),
            pl.BlockSpec(memory_space=pltpu.SMEM),
            pl.BlockSpec(memory_space=pltpu.SMEM),
        ],
        out_shape=[
            jax.ShapeDtypeStruct((n, 2 * d), jnp.float32),
            jax.ShapeDtypeStruct((n, m), jnp.float32),
            jax.ShapeDtypeStruct((1, m), jnp.float32),
            jax.ShapeDtypeStruct((1, m), jnp.float32),
            jax.ShapeDtypeStruct((1, 1), jnp.float32),
            jax.ShapeDtypeStruct((1, 1), jnp.float32),
        ],
    )(qt, kt)

    sq, um = pl.pallas_call(
        _pass2,
        grid=(nt,),
        in_specs=[
            pl.BlockSpec((tile, 2 * d), lambda i: (i, 0)),
            pl.BlockSpec((d, m), lambda i: (0, 0)),
            pl.BlockSpec((m, d), lambda i: (0, 0)),
            pl.BlockSpec((1, m), lambda i: (0, 0)),
            pl.BlockSpec((1, m), lambda i: (0, 0)),
        ],
        out_specs=[
            pl.BlockSpec((tile, m), lambda i: (i, 0)),
            pl.BlockSpec((m, d), lambda i: (0, 0)),
        ],
        out_shape=[
            jax.ShapeDtypeStruct((n, m), jnp.float32),
            jax.ShapeDtypeStruct((m, d), jnp.float32),
        ],
    )(cat, kt, keys, colmax, colsum)

    uq = jnp.transpose(cat.reshape(b, h, w, 2 * d), (0, 3, 1, 2))
    uo = jnp.transpose(cat[:, d:].reshape(b, h, w, d), (0, 3, 1, 2))
    return (uq, uo, um, sq, sm, sep.reshape(()), comp.reshape(()))


# speedup vs baseline: 1.2763x; 1.1713x over previous
import functools

import jax
import jax.numpy as jnp
from jax.experimental import pallas as pl
from jax.experimental.pallas import tpu as pltpu


def _pass1(qt_ref, kt_ref, keys_ref, cat_ref, sm_ref, colmax_ref, colsum_ref,
           comp_ref, sep_ref, *, n_total):
    i = pl.program_id(0)
    nt = pl.num_programs(0)
    q = qt_ref[...]
    kt = kt_ref[...]
    keys = keys_ref[...]
    t, d = q.shape
    m = kt.shape[1]

    qs2 = jnp.sum(q * q, axis=1, keepdims=True)
    qn = q * jax.lax.rsqrt(jnp.maximum(qs2, 1e-24))
    qnn = jnp.sum(qn * qn, axis=1, keepdims=True)
    qs = jnp.sum(qn, axis=1, keepdims=True)

    kn2c = jnp.sum(keys * keys, axis=1, keepdims=True)
    ksc = jnp.sum(keys, axis=1, keepdims=True)

    score = jnp.dot(qn, kt, preferred_element_type=jnp.float32)

    rmax = jnp.max(score, axis=1, keepdims=True)
    er = jnp.exp(score - rmax)
    k1 = jnp.concatenate([keys, jnp.ones((m, 1), jnp.float32)], axis=1)
    ercm = jnp.dot(er, k1, preferred_element_type=jnp.float32)
    inv = 1.0 / ercm[:, d:d + 1]
    smv = er * inv
    sm_ref[...] = smv
    cm = ercm[:, :d] * inv
    cat_ref[:, :d] = qn
    cat_ref[:, d:] = cm

    oh1 = score == rmax
    ohf1 = jnp.where(oh1, 1.0, 0.0)
    g1 = jnp.dot(ohf1, jnp.concatenate([kn2c, ksc], axis=1),
                 preferred_element_type=jnp.float32)
    kn2g = g1[:, 0:1]
    ksumg = g1[:, 1:2]
    masked = jnp.where(oh1, -jnp.inf, score)
    m2 = jnp.max(masked, axis=1, keepdims=True)
    oh2 = masked == m2
    cg2 = jnp.dot(jnp.where(oh2, 1.0, 0.0), kn2c - 2e-6 * ksc,
                  preferred_element_type=jnp.float32)

    base = qnn + 2e-6 * qs + 6.4e-11
    comp_part = jnp.sum(qnn - 2.0 * rmax + kn2g)
    dp = jnp.sqrt(jnp.maximum(base - 2.0 * rmax + kn2g - 2e-6 * ksumg, 0.0))
    dn = jnp.sqrt(jnp.maximum(base - 2.0 * m2 + cg2, 0.0))
    sep_part = jnp.sum(jnp.maximum(dp - dn + 1.0, 0.0))

    eS = er * jnp.exp(rmax)
    ctile_max = jnp.max(score, axis=0, keepdims=True)
    ctile_sum = jnp.sum(eS, axis=0, keepdims=True)

    @pl.when(i == 0)
    def _():
        colmax_ref[...] = jnp.full((1, m), -jnp.inf, jnp.float32)
        colsum_ref[...] = jnp.zeros((1, m), jnp.float32)
        comp_ref[0, 0] = 0.0
        sep_ref[0, 0] = 0.0

    colmax_ref[...] = jnp.maximum(colmax_ref[...], ctile_max)
    colsum_ref[...] = colsum_ref[...] + ctile_sum
    comp_ref[0, 0] = comp_ref[0, 0] + comp_part
    sep_ref[0, 0] = sep_ref[0, 0] + sep_part

    @pl.when(i == nt - 1)
    def _():
        comp_ref[0, 0] = comp_ref[0, 0] / float(n_total * d)
        sep_ref[0, 0] = sep_ref[0, 0] / float(n_total)


def _pass2(cat_ref, kt_ref, keys_ref, colmax_ref, colsum_ref, sq_ref, um_ref):
    i = pl.program_id(0)
    nt = pl.num_programs(0)
    kt = kt_ref[...]
    d, m = kt.shape
    qn = cat_ref[:, :d]

    score = jnp.dot(qn, kt, preferred_element_type=jnp.float32)
    e = jnp.exp(score)
    sq_ref[...] = e * (1.0 / colsum_ref[...])

    rmax = jnp.max(score, axis=1, keepdims=True)
    oh1 = score == rmax
    cmaxg = jnp.sum(jnp.where(oh1, colmax_ref[...], 0.0),
                    axis=1, keepdims=True)
    wgt = jnp.exp(rmax - cmaxg)
    ohf = jnp.where(oh1, 1.0, 0.0)
    wq = wgt * qn
    part = jax.lax.dot_general(ohf, wq, (((0,), (0,)), ((), ())),
                               preferred_element_type=jnp.float32)

    @pl.when(i == 0)
    def _():
        um_ref[...] = jnp.zeros((m, d), jnp.float32)

    um_ref[...] = um_ref[...] + part

    @pl.when(i == nt - 1)
    def _():
        um = um_ref[...] + keys_ref[...]
        nrm = jnp.sqrt(jnp.sum(um * um, axis=1, keepdims=True))
        um_ref[...] = um / jnp.maximum(nrm, 1e-12)


def kernel(query, keys):
    b, d, h, w = query.shape
    m = keys.shape[0]
    n = b * h * w
    qt = jnp.transpose(query, (0, 2, 3, 1)).reshape(n, d)
    kt = keys.T

    tile = 2048
    nt = n // tile

    cat, sm, colmax, colsum, comp, sep = pl.pallas_call(
        functools.partial(_pass1, n_total=n),
        grid=(nt,),
        in_specs=[
            pl.BlockSpec((tile, d), lambda i: (i, 0)),
            pl.BlockSpec((d, m), lambda i: (0, 0)),
            pl.BlockSpec((m, d), lambda i: (0, 0)),
        ],
        out_specs=[
            pl.BlockSpec((tile, 2 * d), lambda i: (i, 0)),
            pl.BlockSpec((tile, m), lambda i: (i, 0)),
            pl.BlockSpec((1, m), lambda i: (0, 0)),
            pl.BlockSpec((1, m), lambda i: (0, 0)),
            pl.BlockSpec(memory_space=pltpu.SMEM),
            pl.BlockSpec(memory_space=pltpu.SMEM),
        ],
        out_shape=[
            jax.ShapeDtypeStruct((n, 2 * d), jnp.float32),
            jax.ShapeDtypeStruct((n, m), jnp.float32),
            jax.ShapeDtypeStruct((1, m), jnp.float32),
            jax.ShapeDtypeStruct((1, m), jnp.float32),
            jax.ShapeDtypeStruct((1, 1), jnp.float32),
            jax.ShapeDtypeStruct((1, 1), jnp.float32),
        ],
    )(qt, kt, keys)

    sq, um = pl.pallas_call(
        _pass2,
        grid=(nt,),
        in_specs=[
            pl.BlockSpec((tile, 2 * d), lambda i: (i, 0)),
            pl.BlockSpec((d, m), lambda i: (0, 0)),
            pl.BlockSpec((m, d), lambda i: (0, 0)),
            pl.BlockSpec((1, m), lambda i: (0, 0)),
            pl.BlockSpec((1, m), lambda i: (0, 0)),
        ],
        out_specs=[
            pl.BlockSpec((tile, m), lambda i: (i, 0)),
            pl.BlockSpec((m, d), lambda i: (0, 0)),
        ],
        out_shape=[
            jax.ShapeDtypeStruct((n, m), jnp.float32),
            jax.ShapeDtypeStruct((m, d), jnp.float32),
        ],
    )(cat, kt, keys, colmax, colsum)

    uq = jnp.transpose(cat.reshape(b, h, w, 2 * d), (0, 3, 1, 2))
    uo = jnp.transpose(cat[:, d:].reshape(b, h, w, d), (0, 3, 1, 2))
    return (uq, uo, um, sq, sm, sep.reshape(()), comp.reshape(()))
